# Initial kernel scaffold; baseline (speedup 1.0000x reference)
import jax
import jax.numpy as jnp
from jax.experimental import pallas as pl


def kernel(encoded_nodes, cluster_ids, num_clusters, W, b):
    raise NotImplementedError("write your pallas kernel here")


# speedup vs baseline: 12.4956x; 12.4956x over previous
import functools

import jax
import jax.numpy as jnp
from jax import lax
from jax.experimental import pallas as pl
from jax.experimental.pallas import tpu as pltpu
import jax.experimental.pallas.tpu_sc as plsc

B, P, E = 16, 4096, 128
NUM_SEGMENTS = 100
CP = 128
NC, NS, L = 2, 16, 16
NW = NC * NS
HALF = P // 2
CHUNK = 128
NCHUNK = HALF // CHUNK

_mesh = plsc.VectorSubcoreMesh(core_axis_name="c", subcore_axis_name="s")


def _worker(cid, sid):
    wid = sid * NC + cid
    bat = wid // 2
    half = wid % 2
    return wid, bat, bat * P + half * HALF


@functools.partial(
    pl.kernel,
    out_type=jax.ShapeDtypeStruct((NW, CP, E), jnp.float32),
    mesh=_mesh,
    scratch_types=[
        pltpu.VMEM((NCHUNK, CHUNK), jnp.int32),
        pltpu.VMEM((2, CHUNK, E), jnp.float32),
        pltpu.VMEM((CP, E), jnp.float32),
        pltpu.VMEM_SHARED((NS * CP, E), jnp.float32),
        pltpu.SemaphoreType.DMA,
        pltpu.SemaphoreType.DMA,
    ],
)
def _seg_sum(nodes_hbm, ids_hbm, out_hbm, ids_v, buf, zbuf, acc_sh, sem0, sem1):
    cid, sid = lax.axis_index("c"), lax.axis_index("s")
    wid, _, base_row = _worker(cid, sid)
    sems = (sem0, sem1)
    pltpu.sync_copy(ids_hbm.at[wid], ids_v)
    for g in range(2):
        pltpu.async_copy(
            nodes_hbm.at[pl.ds(base_row + g * CHUNK, CHUNK)], buf.at[g], sems[g])
    zero16 = jnp.zeros((L,), jnp.float32)

    def _zrow(r, carry):
        for j in range(E // L):
            zbuf[r, pl.ds(j * L, L)] = zero16
        return carry

    lax.fori_loop(0, CP, _zrow, 0)
    row_off = (sid * CP).astype(jnp.int32)
    pltpu.sync_copy(zbuf, acc_sh.at[pl.ds(row_off, CP)])

    def _arow(r, carry):
        for j in range(CHUNK // L):
            ids_v[r, pl.ds(j * L, L)] = ids_v[r, pl.ds(j * L, L)] + row_off
        return carry

    lax.fori_loop(0, NCHUNK, _arow, 0)
    for g in range(NCHUNK):
        s = g % 2
        pltpu.make_async_copy(
            nodes_hbm.at[pl.ds(base_row + g * CHUNK, CHUNK)], buf.at[s], sems[s]
        ).wait()
        pltpu.sync_copy(buf.at[s], acc_sh.at[ids_v.at[g]], add=True)
        if g + 2 < NCHUNK:
            pltpu.async_copy(
                nodes_hbm.at[pl.ds(base_row + (g + 2) * CHUNK, CHUNK)],
                buf.at[s], sems[s])
    pltpu.sync_copy(acc_sh.at[pl.ds(row_off, CP)], out_hbm.at[wid])


def _tc_body(part_ref, ids_ref, wt_ref, b_ref, out_ref):
    sums = part_ref[0, 0] + part_ref[0, 1]
    ciota = lax.broadcasted_iota(jnp.int32, (CP, 128), 0)
    cnt = jnp.zeros((CP, 128), jnp.float32)
    for k in range(P // 128):
        row = ids_ref[0, k, :]
        cnt = cnt + (row[None, :] == ciota).astype(jnp.float32)
    counts = jnp.sum(cnt, axis=1, keepdims=True)
    means = sums / jnp.maximum(counts, 1.0)
    out_ref[0] = (
        jnp.dot(means, wt_ref[...], preferred_element_type=jnp.float32)
        + b_ref[0][None, :])


def _tc_linear(partials, ids_b, w_t, b_row):
    return pl.pallas_call(
        _tc_body,
        grid=(B,),
        in_specs=[
            pl.BlockSpec((1, 2, CP, E), lambda i: (i, 0, 0, 0)),
            pl.BlockSpec((1, P // 128, 128), lambda i: (i, 0, 0)),
            pl.BlockSpec((E, E), lambda i: (0, 0)),
            pl.BlockSpec((1, E), lambda i: (0, 0)),
        ],
        out_specs=pl.BlockSpec((1, CP, E), lambda i: (i, 0, 0)),
        out_shape=jax.ShapeDtypeStruct((B, CP, E), jnp.float32),
    )(partials, ids_b, w_t, b_row)


@functools.partial(
    pl.kernel,
    out_type=jax.ShapeDtypeStruct((B * P, E), jnp.float32),
    mesh=_mesh,
    scratch_types=[
        pltpu.VMEM((NCHUNK, CHUNK), jnp.int32),
        pltpu.VMEM((2, CHUNK, E), jnp.float32),
        pltpu.SemaphoreType.DMA,
        pltpu.SemaphoreType.DMA,
    ],
)
def _gather_back(embs_hbm, ids_hbm, out_hbm, ids_v, buf, sem0, sem1):
    wid, bat, base_row = _worker(lax.axis_index("c"), lax.axis_index("s"))
    sems = (sem0, sem1)
    pltpu.sync_copy(ids_hbm.at[wid], ids_v)
    base = (bat * CP).astype(jnp.int32)

    def _arow(r, carry):
        for j in range(CHUNK // L):
            ids_v[r, pl.ds(j * L, L)] = ids_v[r, pl.ds(j * L, L)] + base
        return carry

    lax.fori_loop(0, NCHUNK, _arow, 0)
    pltpu.async_copy(embs_hbm.at[ids_v.at[0]], buf.at[0], sems[0])
    for g in range(NCHUNK):
        s = g % 2
        pltpu.make_async_copy(embs_hbm.at[ids_v.at[g]], buf.at[s], sems[s]).wait()
        if g + 1 < NCHUNK:
            s2 = (g + 1) % 2
            pltpu.async_copy(embs_hbm.at[ids_v.at[g + 1]], buf.at[s2], sems[s2])
        pltpu.sync_copy(buf.at[s], out_hbm.at[pl.ds(base_row + g * CHUNK, CHUNK)])


def kernel(encoded_nodes, cluster_ids, num_clusters, W, b):
    ids = jnp.clip(cluster_ids.astype(jnp.int32), 0, num_clusters - 1)
    ids3 = ids.reshape(NW, NCHUNK, CHUNK)
    nodes_flat = encoded_nodes.reshape(B * P, E)
    partials = _seg_sum(nodes_flat, ids3)
    embs_pad = _tc_linear(
        partials.reshape(B, 2, CP, E),
        ids.reshape(B, P // 128, 128),
        W.T,
        b.reshape(1, E),
    )
    g_flat = _gather_back(embs_pad.reshape(B * CP, E), ids3)
    return embs_pad[:, :NUM_SEGMENTS, :], g_flat.reshape(B, P, E)


# speedup vs baseline: 13.1064x; 1.0489x over previous
import functools

import jax
import jax.numpy as jnp
from jax import lax
from jax.experimental import pallas as pl
from jax.experimental.pallas import tpu as pltpu
import jax.experimental.pallas.tpu_sc as plsc

B, P, E = 16, 4096, 128
NUM_SEGMENTS = 100
CP = 128
NC, NS, L = 2, 16, 16
NW = NC * NS
HALF = P // 2
CHUNK = 128
NCHUNK = HALF // CHUNK

_mesh = plsc.VectorSubcoreMesh(core_axis_name="c", subcore_axis_name="s")


def _worker(cid, sid):
    wid = sid * NC + cid
    bat = wid // 2
    half = wid % 2
    return wid, bat, bat * P + half * HALF


@functools.partial(
    pl.kernel,
    out_type=jax.ShapeDtypeStruct((NW, CP, E), jnp.float32),
    mesh=_mesh,
    scratch_types=[
        pltpu.VMEM((NCHUNK, CHUNK), jnp.int32),
        pltpu.VMEM((4, CHUNK, E), jnp.float32),
        pltpu.VMEM((CP, E), jnp.float32),
        pltpu.VMEM_SHARED((NS * CP, E), jnp.float32),
        pltpu.SemaphoreType.DMA((4,)),
        pltpu.SemaphoreType.DMA((4,)),
    ],
)
def _seg_sum(nodes_hbm, ids_hbm, out_hbm, ids_v, buf, zbuf, acc_sh, gsem, asem):
    cid, sid = lax.axis_index("c"), lax.axis_index("s")
    wid, _, base_row = _worker(cid, sid)

    def _gather(g):
        return pltpu.async_copy(
            nodes_hbm.at[pl.ds(base_row + g * CHUNK, CHUNK)],
            buf.at[g % 4], gsem.at[g % 4])

    pltpu.sync_copy(ids_hbm.at[wid], ids_v)
    fetches = [_gather(g) for g in range(3)]
    zero16 = jnp.zeros((L,), jnp.float32)

    def _zrow(r, carry):
        for j in range(E // L):
            zbuf[r, pl.ds(j * L, L)] = zero16
        return carry

    lax.fori_loop(0, CP, _zrow, 0)
    row_off = (sid * CP).astype(jnp.int32)
    pltpu.sync_copy(zbuf, acc_sh.at[pl.ds(row_off, CP)])

    def _arow(r, carry):
        for j in range(CHUNK // L):
            ids_v[r, pl.ds(j * L, L)] = ids_v[r, pl.ds(j * L, L)] + row_off
        return carry

    lax.fori_loop(0, NCHUNK, _arow, 0)
    adds = []
    for g in range(NCHUNK):
        fetches[g].wait()
        if g >= 1:
            adds[g - 1].wait()
        adds.append(pltpu.async_copy(
            buf.at[g % 4], acc_sh.at[ids_v.at[g]], asem.at[g % 4], add=True))
        if g + 3 < NCHUNK:
            fetches.append(_gather(g + 3))
    adds[NCHUNK - 1].wait()
    pltpu.sync_copy(acc_sh.at[pl.ds(row_off, CP)], out_hbm.at[wid])


def _tc_body(part_ref, ids_ref, wt_ref, b_ref, out_ref):
    sums = part_ref[0, 0] + part_ref[0, 1]
    ciota = lax.broadcasted_iota(jnp.int32, (CP, 128), 0)
    cnt = jnp.zeros((CP, 128), jnp.float32)
    for k in range(P // 128):
        row = ids_ref[0, k, :]
        cnt = cnt + (row[None, :] == ciota).astype(jnp.float32)
    counts = jnp.sum(cnt, axis=1, keepdims=True)
    means = sums / jnp.maximum(counts, 1.0)
    out_ref[0] = (
        jnp.dot(means, wt_ref[...], preferred_element_type=jnp.float32)
        + b_ref[0][None, :])


def _tc_linear(partials, ids_b, w_t, b_row):
    return pl.pallas_call(
        _tc_body,
        grid=(B,),
        in_specs=[
            pl.BlockSpec((1, 2, CP, E), lambda i: (i, 0, 0, 0)),
            pl.BlockSpec((1, P // 128, 128), lambda i: (i, 0, 0)),
            pl.BlockSpec((E, E), lambda i: (0, 0)),
            pl.BlockSpec((1, E), lambda i: (0, 0)),
        ],
        out_specs=pl.BlockSpec((1, CP, E), lambda i: (i, 0, 0)),
        out_shape=jax.ShapeDtypeStruct((B, CP, E), jnp.float32),
    )(partials, ids_b, w_t, b_row)


@functools.partial(
    pl.kernel,
    out_type=jax.ShapeDtypeStruct((B * P, E), jnp.float32),
    mesh=_mesh,
    scratch_types=[
        pltpu.VMEM((NCHUNK, CHUNK), jnp.int32),
        pltpu.VMEM((4, CHUNK, E), jnp.float32),
        pltpu.SemaphoreType.DMA((4,)),
        pltpu.SemaphoreType.DMA((4,)),
    ],
)
def _gather_back(embs_hbm, ids_hbm, out_hbm, ids_v, buf, gsem, wsem):
    wid, bat, base_row = _worker(lax.axis_index("c"), lax.axis_index("s"))
    pltpu.sync_copy(ids_hbm.at[wid], ids_v)
    base = (bat * CP).astype(jnp.int32)

    def _arow(r, carry):
        for j in range(CHUNK // L):
            ids_v[r, pl.ds(j * L, L)] = ids_v[r, pl.ds(j * L, L)] + base
        return carry

    lax.fori_loop(0, NCHUNK, _arow, 0)

    def _gather(g):
        return pltpu.async_copy(
            embs_hbm.at[ids_v.at[g]], buf.at[g % 4], gsem.at[g % 4])

    fetches = [_gather(g) for g in range(3)]
    writes = []
    for g in range(NCHUNK):
        fetches[g].wait()
        writes.append(pltpu.async_copy(
            buf.at[g % 4], out_hbm.at[pl.ds(base_row + g * CHUNK, CHUNK)],
            wsem.at[g % 4]))
        if g >= 1:
            writes[g - 1].wait()
        if g + 3 < NCHUNK:
            fetches.append(_gather(g + 3))
    writes[NCHUNK - 1].wait()


def kernel(encoded_nodes, cluster_ids, num_clusters, W, b):
    ids = jnp.clip(cluster_ids.astype(jnp.int32), 0, num_clusters - 1)
    ids3 = ids.reshape(NW, NCHUNK, CHUNK)
    nodes_flat = encoded_nodes.reshape(B * P, E)
    partials = _seg_sum(nodes_flat, ids3)
    embs_pad = _tc_linear(
        partials.reshape(B, 2, CP, E),
        ids.reshape(B, P // 128, 128),
        W.T,
        b.reshape(1, E),
    )
    g_flat = _gather_back(embs_pad.reshape(B * CP, E), ids3)
    return embs_pad[:, :NUM_SEGMENTS, :], g_flat.reshape(B, P, E)


# speedup vs baseline: 14.2635x; 1.0883x over previous
import functools

import jax
import jax.numpy as jnp
from jax import lax
from jax.experimental import pallas as pl
from jax.experimental.pallas import tpu as pltpu
import jax.experimental.pallas.tpu_sc as plsc

B, P, E = 16, 4096, 128
NUM_SEGMENTS = 100
CP = 128
NC, NS, L = 2, 16, 16
NW = NC * NS
HALF = P // 2
CHUNK = 128
NCHUNK = HALF // CHUNK

_mesh = plsc.VectorSubcoreMesh(core_axis_name="c", subcore_axis_name="s")


def _worker(cid, sid):
    wid = sid * NC + cid
    bat = wid // 2
    half = wid % 2
    return wid, bat, bat * P + half * HALF


@functools.partial(
    pl.kernel,
    out_type=jax.ShapeDtypeStruct((NW, CP, E), jnp.float32),
    mesh=_mesh,
    scratch_types=[
        pltpu.VMEM((NCHUNK, CHUNK), jnp.int32),
        pltpu.VMEM((CHUNK,), jnp.int32),
        pltpu.VMEM((2 * CHUNK, E), jnp.float32),
        pltpu.VMEM((CP, E), jnp.float32),
        pltpu.VMEM_SHARED((NS * CP, E), jnp.float32),
    ],
)
def _seg_sum(nodes_hbm, ids_hbm, out_hbm, ids_v, idxb, buf, zbuf, acc_sh):
    cid, sid = lax.axis_index("c"), lax.axis_index("s")
    wid, _, base_row = _worker(cid, sid)
    pltpu.sync_copy(ids_hbm.at[wid], ids_v)
    zero16 = jnp.zeros((L,), jnp.float32)

    def _zrow(r, carry):
        for j in range(E // L):
            zbuf[r, pl.ds(j * L, L)] = zero16
        return carry

    lax.fori_loop(0, CP, _zrow, 0)
    row_off = (sid * CP).astype(jnp.int32)
    pltpu.sync_copy(zbuf, acc_sh.at[pl.ds(row_off, CP)])
    for gg in range(NCHUNK // 2):
        pltpu.sync_copy(
            nodes_hbm.at[pl.ds(base_row + gg * 2 * CHUNK, 2 * CHUNK)], buf)
        for h in range(2):
            g = gg * 2 + h
            for j in range(CHUNK // L):
                idxb[pl.ds(j * L, L)] = ids_v[g, pl.ds(j * L, L)] + row_off
            pltpu.sync_copy(
                buf.at[pl.ds(h * CHUNK, CHUNK)], acc_sh.at[idxb], add=True)
    pltpu.sync_copy(acc_sh.at[pl.ds(row_off, CP)], out_hbm.at[wid])


def _tc_body(part_ref, ids_ref, wt_ref, b_ref, out_ref):
    sums = part_ref[0, 0] + part_ref[0, 1]
    ciota = lax.broadcasted_iota(jnp.int32, (CP, 128), 0)
    cnt = jnp.zeros((CP, 128), jnp.float32)
    for k in range(P // 128):
        row = ids_ref[0, k, :]
        cnt = cnt + (row[None, :] == ciota).astype(jnp.float32)
    counts = jnp.sum(cnt, axis=1, keepdims=True)
    means = sums / jnp.maximum(counts, 1.0)
    out_ref[0] = (
        jnp.dot(means, wt_ref[...], preferred_element_type=jnp.float32)
        + b_ref[0][None, :])


def _tc_linear(partials, ids_b, w_t, b_row):
    return pl.pallas_call(
        _tc_body,
        grid=(B,),
        in_specs=[
            pl.BlockSpec((1, 2, CP, E), lambda i: (i, 0, 0, 0)),
            pl.BlockSpec((1, P // 128, 128), lambda i: (i, 0, 0)),
            pl.BlockSpec((E, E), lambda i: (0, 0)),
            pl.BlockSpec((1, E), lambda i: (0, 0)),
        ],
        out_specs=pl.BlockSpec((1, CP, E), lambda i: (i, 0, 0)),
        out_shape=jax.ShapeDtypeStruct((B, CP, E), jnp.float32),
    )(partials, ids_b, w_t, b_row)


@functools.partial(
    pl.kernel,
    out_type=jax.ShapeDtypeStruct((B * P, E), jnp.float32),
    mesh=_mesh,
    scratch_types=[
        pltpu.VMEM((NCHUNK, CHUNK), jnp.int32),
        pltpu.VMEM((4, CHUNK, E), jnp.float32),
        pltpu.VMEM_SHARED((B * CP, E), jnp.float32),
        pltpu.SemaphoreType.DMA((4,)),
        pltpu.SemaphoreType.DMA((4,)),
    ],
)
def _gather_back(embs_hbm, ids_hbm, out_hbm, ids_v, buf, table_sh, gsem, wsem):
    cid, sid = lax.axis_index("c"), lax.axis_index("s")
    wid, bat, base_row = _worker(cid, sid)
    trows = B * CP // NS
    toff = sid * trows
    pltpu.sync_copy(embs_hbm.at[pl.ds(toff, trows)], table_sh.at[pl.ds(toff, trows)])
    pltpu.sync_copy(ids_hbm.at[wid], ids_v)
    base = (bat * CP).astype(jnp.int32)

    def _arow(r, carry):
        for j in range(CHUNK // L):
            ids_v[r, pl.ds(j * L, L)] = ids_v[r, pl.ds(j * L, L)] + base
        return carry

    lax.fori_loop(0, NCHUNK, _arow, 0)
    plsc.subcore_barrier()

    def _gather(g):
        return pltpu.async_copy(
            table_sh.at[ids_v.at[g]], buf.at[g % 4], gsem.at[g % 4])

    fetches = [_gather(g) for g in range(3)]
    writes = []
    for g in range(NCHUNK):
        fetches[g].wait()
        writes.append(pltpu.async_copy(
            buf.at[g % 4], out_hbm.at[pl.ds(base_row + g * CHUNK, CHUNK)],
            wsem.at[g % 4]))
        if g >= 1:
            writes[g - 1].wait()
        if g + 3 < NCHUNK:
            fetches.append(_gather(g + 3))
    writes[NCHUNK - 1].wait()


def kernel(encoded_nodes, cluster_ids, num_clusters, W, b):
    ids = jnp.clip(cluster_ids.astype(jnp.int32), 0, num_clusters - 1)
    ids3 = ids.reshape(NW, NCHUNK, CHUNK)
    nodes_flat = encoded_nodes.reshape(B * P, E)
    partials = _seg_sum(nodes_flat, ids3)
    embs_pad = _tc_linear(
        partials.reshape(B, 2, CP, E),
        ids.reshape(B, P // 128, 128),
        W.T,
        b.reshape(1, E),
    )
    g_flat = _gather_back(embs_pad.reshape(B * CP, E), ids3)
    return embs_pad[:, :NUM_SEGMENTS, :], g_flat.reshape(B, P, E)
